# Initial kernel scaffold; baseline (speedup 1.0000x reference)
import jax
import jax.numpy as jnp
from jax.experimental import pallas as pl


def kernel(derivation_depths, inference_types, parent_counts, depth_pe, embed_table, Wp, bp, Wo, bo, gamma, beta):
    raise NotImplementedError("write your pallas kernel here")


# speedup vs baseline: 8.7639x; 8.7639x over previous
import functools

import jax
import jax.numpy as jnp
from jax import lax
from jax.experimental import pallas as pl
from jax.experimental.pallas import tpu as pltpu
from jax.experimental.pallas import tpu_sc as plsc

_N = 1048576
_D = 64
_ND = 101
_NT = 22
_NP = 8
_TBL = _ND * _NT * _NP

_NW = 32
_RPW = _N // _NW
_C = 512
_NCHUNK = _RPW // _C


def _table_body(pe_ref, emb_ref, wp_ref, bp_ref, wo_ref, bo_ref, g_ref, b_ref,
                out_ref):
    wo = wo_ref[:]
    a = jnp.dot(pe_ref[:], wo[0:32, :], preferred_element_type=jnp.float32)
    b = jnp.dot(emb_ref[:], wo[32:48, :], preferred_element_type=jnp.float32)
    wp_o = jnp.dot(wp_ref[:], wo[48:64, :], preferred_element_type=jnp.float32)
    base = (jnp.dot(bp_ref[:], wo[48:64, :], preferred_element_type=jnp.float32)
            + bo_ref[:])
    pvals = lax.broadcasted_iota(jnp.int32, (_NP, 1), 0).astype(jnp.float32)
    c = pvals * wp_o + base
    x = (a[:, None, None, :] + b[None, :, None, :] + c[None, None, :, :])
    mean = jnp.mean(x, axis=-1, keepdims=True)
    xc = x - mean
    var = jnp.mean(xc * xc, axis=-1, keepdims=True)
    out_ref[:] = xc * lax.rsqrt(var + 1e-5) * g_ref[:] + b_ref[:]


def _build_table(depth_pe, embed_table, Wp, bp, Wo, bo, gamma, beta):
    table4 = pl.pallas_call(
        _table_body,
        out_shape=jax.ShapeDtypeStruct((_ND, _NT, _NP, _D), jnp.float32),
    )(depth_pe, embed_table, Wp, bp.reshape(1, -1), Wo, bo.reshape(1, -1),
      gamma.reshape(1, -1), beta.reshape(1, -1))
    return table4.reshape(_TBL, _D)


def _gather_body(d_hbm, t_hbm, p_hbm, table_hbm, out_hbm,
                 d_v, t_v, p_v, idx_v, rows_v, sem):
    wid = lax.axis_index("s") * 2 + lax.axis_index("c")
    base = wid * _RPW

    def chunk(ci, carry):
        off = base + ci * _C
        pltpu.sync_copy(d_hbm.at[pl.ds(off, _C)], d_v)
        pltpu.sync_copy(t_hbm.at[pl.ds(off, _C)], t_v)
        pltpu.sync_copy(p_hbm.at[pl.ds(off, _C)], p_v)
        for i in range(_C // 16):
            s = pl.ds(i * 16, 16)
            d = jnp.clip(d_v[s], 0, _ND - 1)
            t = jnp.clip(t_v[s], 0, _NT - 1)
            idx_v[s] = d * (_NT * _NP) + t * _NP + p_v[s]
        pltpu.async_copy(table_hbm.at[idx_v], rows_v, sem).wait()
        pltpu.sync_copy(rows_v, out_hbm.at[pl.ds(off, _C)])
        return carry

    lax.fori_loop(0, _NCHUNK, chunk, 0)


@functools.cache
def _make_gather():
    return functools.partial(
        pl.kernel,
        out_type=jax.ShapeDtypeStruct((_N, _D), jnp.float32),
        mesh=plsc.VectorSubcoreMesh(core_axis_name="c", subcore_axis_name="s",
                                    num_cores=2, num_subcores=16),
        scratch_types=[
            pltpu.VMEM((_C,), jnp.int32),
            pltpu.VMEM((_C,), jnp.int32),
            pltpu.VMEM((_C,), jnp.int32),
            pltpu.VMEM((_C,), jnp.int32),
            pltpu.VMEM((_C, _D), jnp.float32),
            pltpu.SemaphoreType.DMA,
        ],
        compiler_params=pltpu.CompilerParams(use_tc_tiling_on_sc=False),
    )(_gather_body)


def kernel(derivation_depths, inference_types, parent_counts, depth_pe,
           embed_table, Wp, bp, Wo, bo, gamma, beta):
    table = _build_table(depth_pe, embed_table, Wp, bp, Wo, bo, gamma, beta)
    return _make_gather()(derivation_depths, inference_types, parent_counts,
                          table)


# speedup vs baseline: 9.3171x; 1.0631x over previous
import functools

import jax
import jax.numpy as jnp
from jax import lax
from jax.experimental import pallas as pl
from jax.experimental.pallas import tpu as pltpu
from jax.experimental.pallas import tpu_sc as plsc

_N = 1048576
_D = 64
_ND = 101
_NT = 22
_NP = 8
_TBL = _ND * _NT * _NP

_NW = 32
_RPW = _N // _NW
_C = 512
_NCHUNK = _RPW // _C


def _table_body(pe_ref, emb_ref, wp_ref, bp_ref, wo_ref, bo_ref, g_ref, b_ref,
                out_ref):
    wo = wo_ref[:]
    a = jnp.dot(pe_ref[:], wo[0:32, :], preferred_element_type=jnp.float32)
    b = jnp.dot(emb_ref[:], wo[32:48, :], preferred_element_type=jnp.float32)
    wp_o = jnp.dot(wp_ref[:], wo[48:64, :], preferred_element_type=jnp.float32)
    base = (jnp.dot(bp_ref[:], wo[48:64, :], preferred_element_type=jnp.float32)
            + bo_ref[:])
    pvals = lax.broadcasted_iota(jnp.int32, (_NP, 1), 0).astype(jnp.float32)
    c = pvals * wp_o + base
    x = (a[:, None, None, :] + b[None, :, None, :] + c[None, None, :, :])
    mean = jnp.mean(x, axis=-1, keepdims=True)
    xc = x - mean
    var = jnp.mean(xc * xc, axis=-1, keepdims=True)
    out_ref[:] = xc * lax.rsqrt(var + 1e-5) * g_ref[:] + b_ref[:]


def _build_table(depth_pe, embed_table, Wp, bp, Wo, bo, gamma, beta):
    table4 = pl.pallas_call(
        _table_body,
        out_shape=jax.ShapeDtypeStruct((_ND, _NT, _NP, _D), jnp.float32),
    )(depth_pe, embed_table, Wp, bp.reshape(1, -1), Wo, bo.reshape(1, -1),
      gamma.reshape(1, -1), beta.reshape(1, -1))
    return table4.reshape(_TBL, _D)


def _gather_body(d_hbm, t_hbm, p_hbm, table_hbm, out_hbm,
                 d_v, t_v, p_v, idx_v, rows_v, sem):
    wid = lax.axis_index("s") * 2 + lax.axis_index("c")
    base = wid * _RPW
    last = _NCHUNK - 1

    def load_idx(slot, ci):
        off = base + ci * _C
        pltpu.sync_copy(d_hbm.at[pl.ds(off, _C)], d_v)
        pltpu.sync_copy(t_hbm.at[pl.ds(off, _C)], t_v)
        pltpu.sync_copy(p_hbm.at[pl.ds(off, _C)], p_v)
        for i in range(_C // 16):
            s = pl.ds(i * 16, 16)
            d = jnp.clip(d_v[s], 0, _ND - 1)
            t = jnp.clip(t_v[s], 0, _NT - 1)
            idx_v[slot, s] = d * (_NT * _NP) + t * _NP + p_v[s]

    def start_gather(slot):
        return pltpu.async_copy(table_hbm.at[idx_v.at[slot]],
                                rows_v.at[slot], sem)

    def wait_gather(slot):
        pltpu.make_async_copy(table_hbm.at[idx_v.at[slot]],
                              rows_v.at[slot], sem).wait()

    load_idx(0, 0)
    start_gather(0)
    load_idx(1, 1)

    def outer(gi, carry):
        for b in (0, 1):
            g = 2 * gi + b
            nb = 1 - b
            wait_gather(b)
            start_gather(nb)
            pltpu.sync_copy(rows_v.at[b], out_hbm.at[pl.ds(base + g * _C, _C)])
            load_idx(b, jnp.minimum(g + 2, last))
        return carry

    lax.fori_loop(0, _NCHUNK // 2, outer, 0)
    wait_gather(0)


@functools.cache
def _make_gather():
    return functools.partial(
        pl.kernel,
        out_type=jax.ShapeDtypeStruct((_N, _D), jnp.float32),
        mesh=plsc.VectorSubcoreMesh(core_axis_name="c", subcore_axis_name="s",
                                    num_cores=2, num_subcores=16),
        scratch_types=[
            pltpu.VMEM((_C,), jnp.int32),
            pltpu.VMEM((_C,), jnp.int32),
            pltpu.VMEM((_C,), jnp.int32),
            pltpu.VMEM((2, _C), jnp.int32),
            pltpu.VMEM((2, _C, _D), jnp.float32),
            pltpu.SemaphoreType.DMA,
        ],
        compiler_params=pltpu.CompilerParams(use_tc_tiling_on_sc=False),
    )(_gather_body)


def kernel(derivation_depths, inference_types, parent_counts, depth_pe,
           embed_table, Wp, bp, Wo, bo, gamma, beta):
    table = _build_table(depth_pe, embed_table, Wp, bp, Wo, bo, gamma, beta)
    return _make_gather()(derivation_depths, inference_types, parent_counts,
                          table)
